# baseline (speedup 1.0000x reference)
import functools

import jax
import jax.numpy as jnp
from jax import lax
from jax.experimental import pallas as pl
from jax.experimental.pallas import tpu as pltpu
from jax.experimental.pallas import tpu_sc as plsc

EMBED = 768
NH = 12
HD = 64
NP = 4
GH = 64
GW = 64
HWG = GH * GW
EPS = 1e-6


_MQ = 4


_HWB = 2048


def _vtab_kernel(x_ref, wn_ref, bn_ref, w_ref, b_ref, o_ref, xn_scr):
    mq = pl.program_id(2)

    @pl.when(mq == 0)
    def _():
        x = x_ref[...]
        mu = jnp.mean(x, axis=-1, keepdims=True)
        var = jnp.mean((x - mu) ** 2, axis=-1, keepdims=True)
        xn_scr[...] = (x - mu) * lax.rsqrt(var + EPS) * wn_ref[...] + bn_ref[...]

    v = jnp.dot(xn_scr[...], w_ref[...], preferred_element_type=jnp.float32) + b_ref[...]
    for h in range(_MQ):
        o_ref[h] = v[:, h * HD:(h + 1) * HD]


def _value_table(kv2d, wn, bn, wT, b):
    rows = kv2d.shape[0]
    nbat = rows // HWG
    nh2 = HWG // _HWB
    nmq = NH // _MQ
    out = pl.pallas_call(
        _vtab_kernel,
        grid=(nbat, nh2, nmq),
        in_specs=[
            pl.BlockSpec((_HWB, EMBED), lambda b_, h_, m_: (b_ * nh2 + h_, 0)),
            pl.BlockSpec((1, EMBED), lambda b_, h_, m_: (0, 0)),
            pl.BlockSpec((1, EMBED), lambda b_, h_, m_: (0, 0)),
            pl.BlockSpec((EMBED, _MQ * HD), lambda b_, h_, m_: (0, m_)),
            pl.BlockSpec((1, _MQ * HD), lambda b_, h_, m_: (0, m_)),
        ],
        out_specs=pl.BlockSpec((_MQ, _HWB, HD),
                               lambda b_, h_, m_: (b_ * nmq + m_, h_, 0)),
        out_shape=jax.ShapeDtypeStruct((nbat * NH, HWG, HD), jnp.float32),
        scratch_shapes=[pltpu.VMEM((_HWB, EMBED), jnp.float32)],
    )(kv2d, wn.reshape(1, -1), bn.reshape(1, -1), wT, b.reshape(1, -1))
    return out.reshape(rows * NH, HD)


def _qproj_kernel(x_ref, wn_ref, bn_ref, w_ref, b_ref, pool_ref, ref_ref,
                  idx_ref, w_out_ref):
    x = x_ref[...]
    mu = jnp.mean(x, axis=-1, keepdims=True)
    var = jnp.mean((x - mu) ** 2, axis=-1, keepdims=True)
    xn = (x - mu) * lax.rsqrt(var + EPS) * wn_ref[...] + bn_ref[...]
    oa = jnp.dot(xn, w_ref[...], preferred_element_type=jnp.float32) + b_ref[...]
    X, Y, A = oa[:, 0:256], oa[:, 256:512], oa[:, 512:768]

    refs = ref_ref[...]
    px = refs[:, 0:1] * GW + X - 0.5
    py = refs[:, 1:2] * GH + Y - 0.5
    x0 = jnp.floor(px)
    y0 = jnp.floor(py)
    wx1 = px - x0
    wy1 = py - y0

    li = lax.broadcasted_iota(jnp.int32, (1, 256), 1)
    c = li % 4
    cxf = (c % 2).astype(jnp.float32)
    cyf = (c // 2).astype(jnp.float32)
    ix = x0 + cxf
    iy = y0 + cyf
    valid = ((ix >= 0) & (ix <= GW - 1) & (iy >= 0) & (iy <= GH - 1)).astype(jnp.float32)
    wx = jnp.where(cxf > 0, wx1, 1.0 - wx1)
    wy = jnp.where(cyf > 0, wy1, 1.0 - wy1)

    Am = A - jnp.max(A, axis=-1, keepdims=True)
    e = jnp.exp(Am)
    denom = jnp.dot(e, pool_ref[...], preferred_element_type=jnp.float32)
    aw = e / denom
    w_out_ref[...] = aw * wx * wy * valid

    ixc = jnp.clip(ix, 0, GW - 1).astype(jnp.int32)
    iyc = jnp.clip(iy, 0, GH - 1).astype(jnp.int32)
    b_sc = pl.program_id(0) // 2
    base = (b_sc * NH + li // 16) * HWG
    idx_ref[...] = base + iyc * GW + ixc


def _q_projections(q2d, ref2d, wn, bn, W_off, b_off, W_attn, b_attn, blk):
    rows = q2d.shape[0]
    l = jnp.arange(192)
    row_pm = (l // 16) * NP + (l // 4) % 4
    wx_e = jnp.zeros((EMBED, 256), jnp.float32).at[:, :192].set(W_off.T[:, row_pm * 2])
    wy_e = jnp.zeros((EMBED, 256), jnp.float32).at[:, :192].set(W_off.T[:, row_pm * 2 + 1])
    wa_e = jnp.zeros((EMBED, 256), jnp.float32).at[:, :192].set(W_attn.T[:, row_pm])
    w_exp = jnp.concatenate([wx_e, wy_e, wa_e], axis=1)
    bx_e = jnp.zeros((256,), jnp.float32).at[:192].set(b_off[row_pm * 2])
    by_e = jnp.zeros((256,), jnp.float32).at[:192].set(b_off[row_pm * 2 + 1])
    ba_e = jnp.zeros((256,), jnp.float32).at[:192].set(b_attn[row_pm])
    b_exp = jnp.concatenate([bx_e, by_e, ba_e])
    mi = l // 16
    pool = jnp.zeros((256, 256), jnp.float32).at[:192, :192].set(
        (mi[:, None] == mi[None, :]).astype(jnp.float32) * 0.25)

    return pl.pallas_call(
        _qproj_kernel,
        grid=(rows // blk,),
        in_specs=[
            pl.BlockSpec((blk, EMBED), lambda i: (i, 0)),
            pl.BlockSpec((1, EMBED), lambda i: (0, 0)),
            pl.BlockSpec((1, EMBED), lambda i: (0, 0)),
            pl.BlockSpec((EMBED, EMBED), lambda i: (0, 0)),
            pl.BlockSpec((1, EMBED), lambda i: (0, 0)),
            pl.BlockSpec((256, 256), lambda i: (0, 0)),
            pl.BlockSpec((blk, 2), lambda i: (i, 0)),
        ],
        out_specs=[
            pl.BlockSpec((blk, 256), lambda i: (i, 0)),
            pl.BlockSpec((blk, 256), lambda i: (i, 0)),
        ],
        out_shape=[
            jax.ShapeDtypeStruct((rows, 256), jnp.int32),
            jax.ShapeDtypeStruct((rows, 256), jnp.float32),
        ],
    )(q2d, wn.reshape(1, -1), bn.reshape(1, -1), w_exp, b_exp.reshape(1, -1),
      pool, ref2d)


_NW = 32
_CH_ROWS = 8


def _sc_compute_chunk(rows_buf, w_v, g, out_v, o0):
    for r in range(_CH_ROWS):
        accs = [None] * 4
        wrow = w_v[g, pl.ds(r * 16, 16)]
        for k in range(16):
            wk = jnp.full((16,), wrow[k])
            for d in range(4):
                term = wk * rows_buf[r * 16 + k, pl.ds(d * 16, 16)]
                accs[d] = term if accs[d] is None else accs[d] + term
        for d in range(4):
            out_v[o0 + r, pl.ds(d * 16, 16)] = accs[d]


def _sc_gather_body(vt_hbm, idx_hbm, wts_hbm, out_hbm,
                    idx_v, w_v, rows0, rows1, rows2, rows3, out_v0, out_v1,
                    isem, gsem0, gsem1, gsem2, gsem3, osem, osem2):
    nc = 2
    wid = lax.axis_index("s") * nc + lax.axis_index("c")
    chunks = idx_hbm.shape[1]
    base_n = wid * (chunks * _CH_ROWS)

    c1 = pltpu.async_copy(idx_hbm.at[wid], idx_v, isem)
    c1.wait()
    c2 = pltpu.async_copy(wts_hbm.at[wid], w_v, isem)
    c2.wait()

    bufs = (rows0, rows1, rows2, rows3)
    gsems = (gsem0, gsem1, gsem2, gsem3)
    outs = (out_v0, out_v1)
    osems = (osem, osem2)

    for g0 in range(4):
        pltpu.async_copy(vt_hbm.at[idx_v.at[g0]], bufs[g0], gsems[g0])

    def iter8(jj, _):
        for s in range(8):
            g = jj * 8 + s
            buf, sem = bufs[s % 4], gsems[s % 4]
            ov, osm = outs[s % 2], osems[s % 2]
            pltpu.make_async_copy(vt_hbm.at[idx_v.at[g]], buf, sem).wait()
            dst = out_hbm.at[pl.ds(base_n + g * _CH_ROWS, _CH_ROWS)]
            @pl.when(g >= 2)
            def _():
                pltpu.make_async_copy(ov, dst, osm).wait()
            _sc_compute_chunk(buf, w_v, g, ov, 0)
            @pl.when(g + 4 < chunks)
            def _():
                pltpu.async_copy(vt_hbm.at[idx_v.at[g + 4]], buf, sem)
            pltpu.async_copy(ov, dst, osm)
        return ()

    lax.fori_loop(0, chunks // 8, iter8, ())
    pltpu.make_async_copy(out_v0, out_hbm.at[pl.ds(base_n, _CH_ROWS)], osem).wait()
    pltpu.make_async_copy(out_v1, out_hbm.at[pl.ds(base_n, _CH_ROWS)], osem2).wait()


def _sc_gather(vt, idx, wts, nrows):
    chunks = nrows // (_NW * _CH_ROWS)
    idx = idx.reshape(_NW, chunks, _CH_ROWS * 16)
    wts = wts.reshape(_NW, chunks, _CH_ROWS * 16)
    mesh = plsc.VectorSubcoreMesh(core_axis_name="c", subcore_axis_name="s")
    f = pl.kernel(
        _sc_gather_body,
        out_type=jax.ShapeDtypeStruct((nrows, HD), jnp.float32),
        mesh=mesh,
        scratch_types=[
            pltpu.VMEM((chunks, _CH_ROWS * 16), jnp.int32),
            pltpu.VMEM((chunks, _CH_ROWS * 16), jnp.float32),
            pltpu.VMEM((_CH_ROWS * 16, HD), jnp.float32),
            pltpu.VMEM((_CH_ROWS * 16, HD), jnp.float32),
            pltpu.VMEM((_CH_ROWS * 16, HD), jnp.float32),
            pltpu.VMEM((_CH_ROWS * 16, HD), jnp.float32),
            pltpu.VMEM((_CH_ROWS, HD), jnp.float32),
            pltpu.VMEM((_CH_ROWS, HD), jnp.float32),
            pltpu.SemaphoreType.DMA,
            pltpu.SemaphoreType.DMA,
            pltpu.SemaphoreType.DMA,
            pltpu.SemaphoreType.DMA,
            pltpu.SemaphoreType.DMA,
            pltpu.SemaphoreType.DMA,
            pltpu.SemaphoreType.DMA,
        ],
        compiler_params=pltpu.CompilerParams(use_tc_tiling_on_sc=False),
    )
    return f(vt, idx, wts)


def _out_kernel(a_ref, q_ref, w_ref, b_ref, g_ref, o_ref):
    y = jnp.dot(a_ref[...], w_ref[...], preferred_element_type=jnp.float32) + b_ref[...]
    o_ref[...] = q_ref[...] + g_ref[...] * y


def _out_proj(attn2d, q2d, W_out, b_out, gamma, blk):
    rows = q2d.shape[0]
    return pl.pallas_call(
        _out_kernel,
        grid=(rows // blk,),
        in_specs=[
            pl.BlockSpec((blk, EMBED), lambda i: (i, 0)),
            pl.BlockSpec((blk, EMBED), lambda i: (i, 0)),
            pl.BlockSpec((EMBED, EMBED), lambda i: (0, 0)),
            pl.BlockSpec((1, EMBED), lambda i: (0, 0)),
            pl.BlockSpec((1, EMBED), lambda i: (0, 0)),
        ],
        out_specs=pl.BlockSpec((blk, EMBED), lambda i: (i, 0)),
        out_shape=jax.ShapeDtypeStruct((rows, EMBED), jnp.float32),
    )(attn2d, q2d, W_out.T, b_out.reshape(1, -1), gamma.reshape(1, -1))


def kernel(q, reference_points, kv, spatial_shapes, level_start_index,
           w_norm1, b_norm1, w_norm2, b_norm2,
           W_off, b_off, W_attn, b_attn, W_val, b_val, W_out, b_out, gamma):
    B, Lq, C = q.shape
    Lin = kv.shape[1]

    vt = _value_table(kv.reshape(B * Lin, C), w_norm2, b_norm2, W_val.T, b_val)

    idx256, w256 = _q_projections(q.reshape(B * Lq, C),
                                  reference_points.reshape(B * Lq, 2),
                                  w_norm1, b_norm1,
                                  W_off, b_off, W_attn, b_attn, blk=512)
    nrows = B * Lq * NH
    idx = idx256[:, :192].reshape(nrows, 16)
    wts = w256[:, :192].reshape(nrows, 16)

    attn2d = _sc_gather(vt, idx, wts, nrows).reshape(B * Lq, C)

    out = _out_proj(attn2d, q.reshape(B * Lq, C), W_out, b_out, gamma, blk=512)
    return out.reshape(B, Lq, C)


# speedup vs baseline: 1.1716x; 1.1716x over previous
import functools

import jax
import jax.numpy as jnp
from jax import lax
from jax.experimental import pallas as pl
from jax.experimental.pallas import tpu as pltpu
from jax.experimental.pallas import tpu_sc as plsc

EMBED = 768
NH = 12
HD = 64
NP = 4
GH = 64
GW = 64
HWG = GH * GW
EPS = 1e-6


def _vtab_kernel(x_ref, wn_ref, bn_ref, w_ref, b_ref, o_ref):
    x = x_ref[...]
    mu = jnp.mean(x, axis=-1, keepdims=True)
    var = jnp.mean((x - mu) ** 2, axis=-1, keepdims=True)
    xn = (x - mu) * lax.rsqrt(var + EPS) * wn_ref[...] + bn_ref[...]
    o_ref[...] = jnp.dot(xn, w_ref[...], preferred_element_type=jnp.float32) + b_ref[...]


def _value_table(kv2d, wn, bn, wT, b, blk=1024):
    rows = kv2d.shape[0]
    out = pl.pallas_call(
        _vtab_kernel,
        grid=(rows // blk,),
        in_specs=[
            pl.BlockSpec((blk, EMBED), lambda i: (i, 0)),
            pl.BlockSpec((1, EMBED), lambda i: (0, 0)),
            pl.BlockSpec((1, EMBED), lambda i: (0, 0)),
            pl.BlockSpec((EMBED, EMBED), lambda i: (0, 0)),
            pl.BlockSpec((1, EMBED), lambda i: (0, 0)),
        ],
        out_specs=pl.BlockSpec((blk, EMBED), lambda i: (i, 0)),
        out_shape=jax.ShapeDtypeStruct((rows, EMBED), jnp.float32),
    )(kv2d, wn.reshape(1, -1), bn.reshape(1, -1), wT, b.reshape(1, -1))
    return out.reshape(rows * NH, HD)


def _qproj_kernel(x_ref, wn_ref, bn_ref, w_ref, b_ref, pool_ref, ref_ref,
                  idx_ref, w_out_ref):
    x = x_ref[...]
    mu = jnp.mean(x, axis=-1, keepdims=True)
    var = jnp.mean((x - mu) ** 2, axis=-1, keepdims=True)
    xn = (x - mu) * lax.rsqrt(var + EPS) * wn_ref[...] + bn_ref[...]
    oa = jnp.dot(xn, w_ref[...], preferred_element_type=jnp.float32) + b_ref[...]
    X, Y, A = oa[:, 0:256], oa[:, 256:512], oa[:, 512:768]

    refs = ref_ref[...]
    px = refs[:, 0:1] * GW + X - 0.5
    py = refs[:, 1:2] * GH + Y - 0.5
    x0 = jnp.floor(px)
    y0 = jnp.floor(py)
    wx1 = px - x0
    wy1 = py - y0

    li = lax.broadcasted_iota(jnp.int32, (1, 256), 1)
    c = li % 4
    cxf = (c % 2).astype(jnp.float32)
    cyf = (c // 2).astype(jnp.float32)
    ix = x0 + cxf
    iy = y0 + cyf
    valid = ((ix >= 0) & (ix <= GW - 1) & (iy >= 0) & (iy <= GH - 1)).astype(jnp.float32)
    wx = jnp.where(cxf > 0, wx1, 1.0 - wx1)
    wy = jnp.where(cyf > 0, wy1, 1.0 - wy1)

    Am = A - jnp.max(A, axis=-1, keepdims=True)
    e = jnp.exp(Am)
    denom = jnp.dot(e, pool_ref[...], preferred_element_type=jnp.float32)
    aw = e / denom
    w_out_ref[...] = aw * wx * wy * valid

    ixc = jnp.clip(ix, 0, GW - 1).astype(jnp.int32)
    iyc = jnp.clip(iy, 0, GH - 1).astype(jnp.int32)
    b_sc = pl.program_id(0) // 2
    idx_ref[...] = (b_sc * HWG + iyc * GW + ixc) * NH + li // 16


def _q_projections(q2d, ref2d, wn, bn, W_off, b_off, W_attn, b_attn, blk):
    rows = q2d.shape[0]
    l = jnp.arange(192)
    row_pm = (l // 16) * NP + (l // 4) % 4
    wx_e = jnp.zeros((EMBED, 256), jnp.float32).at[:, :192].set(W_off.T[:, row_pm * 2])
    wy_e = jnp.zeros((EMBED, 256), jnp.float32).at[:, :192].set(W_off.T[:, row_pm * 2 + 1])
    wa_e = jnp.zeros((EMBED, 256), jnp.float32).at[:, :192].set(W_attn.T[:, row_pm])
    w_exp = jnp.concatenate([wx_e, wy_e, wa_e], axis=1)
    bx_e = jnp.zeros((256,), jnp.float32).at[:192].set(b_off[row_pm * 2])
    by_e = jnp.zeros((256,), jnp.float32).at[:192].set(b_off[row_pm * 2 + 1])
    ba_e = jnp.zeros((256,), jnp.float32).at[:192].set(b_attn[row_pm])
    b_exp = jnp.concatenate([bx_e, by_e, ba_e])
    mi = l // 16
    pool = jnp.zeros((256, 256), jnp.float32).at[:192, :192].set(
        (mi[:, None] == mi[None, :]).astype(jnp.float32) * 0.25)

    return pl.pallas_call(
        _qproj_kernel,
        grid=(rows // blk,),
        in_specs=[
            pl.BlockSpec((blk, EMBED), lambda i: (i, 0)),
            pl.BlockSpec((1, EMBED), lambda i: (0, 0)),
            pl.BlockSpec((1, EMBED), lambda i: (0, 0)),
            pl.BlockSpec((EMBED, EMBED), lambda i: (0, 0)),
            pl.BlockSpec((1, EMBED), lambda i: (0, 0)),
            pl.BlockSpec((256, 256), lambda i: (0, 0)),
            pl.BlockSpec((blk, 2), lambda i: (i, 0)),
        ],
        out_specs=[
            pl.BlockSpec((blk, 256), lambda i: (i, 0)),
            pl.BlockSpec((blk, 256), lambda i: (i, 0)),
        ],
        out_shape=[
            jax.ShapeDtypeStruct((rows, 256), jnp.int32),
            jax.ShapeDtypeStruct((rows, 256), jnp.float32),
        ],
    )(q2d, wn.reshape(1, -1), bn.reshape(1, -1), w_exp, b_exp.reshape(1, -1),
      pool, ref2d)


_NW = 32
_CH_ROWS = 8


def _sc_compute_chunk(rows_buf, w_v, g, out_v, o0):
    for r in range(_CH_ROWS):
        accs = [None] * 4
        wrow = w_v[g, pl.ds(r * 16, 16)]
        for k in range(16):
            wk = jnp.full((16,), wrow[k])
            for d in range(4):
                term = wk * rows_buf[r * 16 + k, pl.ds(d * 16, 16)]
                accs[d] = term if accs[d] is None else accs[d] + term
        for d in range(4):
            out_v[o0 + r, pl.ds(d * 16, 16)] = accs[d]


def _sc_gather_body(vt_hbm, idx_hbm, wts_hbm, out_hbm,
                    idx_v, w_v, rows0, rows1, rows2, rows3, out_v0, out_v1,
                    isem, gsem0, gsem1, gsem2, gsem3, osem, osem2):
    nc = 2
    wid = lax.axis_index("s") * nc + lax.axis_index("c")
    chunks = idx_hbm.shape[1]
    base_n = wid * (chunks * _CH_ROWS)

    c1 = pltpu.async_copy(idx_hbm.at[wid], idx_v, isem)
    c1.wait()
    c2 = pltpu.async_copy(wts_hbm.at[wid], w_v, isem)
    c2.wait()

    bufs = (rows0, rows1, rows2, rows3)
    gsems = (gsem0, gsem1, gsem2, gsem3)
    outs = (out_v0, out_v1)
    osems = (osem, osem2)

    for g0 in range(4):
        pltpu.async_copy(vt_hbm.at[idx_v.at[g0]], bufs[g0], gsems[g0])

    def iter8(jj, _):
        for s in range(8):
            g = jj * 8 + s
            buf, sem = bufs[s % 4], gsems[s % 4]
            ov, osm = outs[s % 2], osems[s % 2]
            pltpu.make_async_copy(vt_hbm.at[idx_v.at[g]], buf, sem).wait()
            dst = out_hbm.at[pl.ds(base_n + g * _CH_ROWS, _CH_ROWS)]
            @pl.when(g >= 2)
            def _():
                pltpu.make_async_copy(ov, dst, osm).wait()
            _sc_compute_chunk(buf, w_v, g, ov, 0)
            @pl.when(g + 4 < chunks)
            def _():
                pltpu.async_copy(vt_hbm.at[idx_v.at[g + 4]], buf, sem)
            pltpu.async_copy(ov, dst, osm)
        return ()

    lax.fori_loop(0, chunks // 8, iter8, ())
    pltpu.make_async_copy(out_v0, out_hbm.at[pl.ds(base_n, _CH_ROWS)], osem).wait()
    pltpu.make_async_copy(out_v1, out_hbm.at[pl.ds(base_n, _CH_ROWS)], osem2).wait()


def _sc_gather(vt, idx, wts, nrows):
    chunks = nrows // (_NW * _CH_ROWS)
    idx = idx.reshape(_NW, chunks, _CH_ROWS * 16)
    wts = wts.reshape(_NW, chunks, _CH_ROWS * 16)
    mesh = plsc.VectorSubcoreMesh(core_axis_name="c", subcore_axis_name="s")
    f = pl.kernel(
        _sc_gather_body,
        out_type=jax.ShapeDtypeStruct((nrows, HD), jnp.float32),
        mesh=mesh,
        scratch_types=[
            pltpu.VMEM((chunks, _CH_ROWS * 16), jnp.int32),
            pltpu.VMEM((chunks, _CH_ROWS * 16), jnp.float32),
            pltpu.VMEM((_CH_ROWS * 16, HD), jnp.float32),
            pltpu.VMEM((_CH_ROWS * 16, HD), jnp.float32),
            pltpu.VMEM((_CH_ROWS * 16, HD), jnp.float32),
            pltpu.VMEM((_CH_ROWS * 16, HD), jnp.float32),
            pltpu.VMEM((_CH_ROWS, HD), jnp.float32),
            pltpu.VMEM((_CH_ROWS, HD), jnp.float32),
            pltpu.SemaphoreType.DMA,
            pltpu.SemaphoreType.DMA,
            pltpu.SemaphoreType.DMA,
            pltpu.SemaphoreType.DMA,
            pltpu.SemaphoreType.DMA,
            pltpu.SemaphoreType.DMA,
            pltpu.SemaphoreType.DMA,
        ],
        compiler_params=pltpu.CompilerParams(use_tc_tiling_on_sc=False),
    )
    return f(vt, idx, wts)


def _out_kernel(a_ref, q_ref, w_ref, b_ref, g_ref, o_ref):
    y = jnp.dot(a_ref[...], w_ref[...], preferred_element_type=jnp.float32) + b_ref[...]
    o_ref[...] = q_ref[...] + g_ref[...] * y


def _out_proj(attn2d, q2d, W_out, b_out, gamma, blk):
    rows = q2d.shape[0]
    return pl.pallas_call(
        _out_kernel,
        grid=(rows // blk,),
        in_specs=[
            pl.BlockSpec((blk, EMBED), lambda i: (i, 0)),
            pl.BlockSpec((blk, EMBED), lambda i: (i, 0)),
            pl.BlockSpec((EMBED, EMBED), lambda i: (0, 0)),
            pl.BlockSpec((1, EMBED), lambda i: (0, 0)),
            pl.BlockSpec((1, EMBED), lambda i: (0, 0)),
        ],
        out_specs=pl.BlockSpec((blk, EMBED), lambda i: (i, 0)),
        out_shape=jax.ShapeDtypeStruct((rows, EMBED), jnp.float32),
    )(attn2d, q2d, W_out.T, b_out.reshape(1, -1), gamma.reshape(1, -1))


def kernel(q, reference_points, kv, spatial_shapes, level_start_index,
           w_norm1, b_norm1, w_norm2, b_norm2,
           W_off, b_off, W_attn, b_attn, W_val, b_val, W_out, b_out, gamma):
    B, Lq, C = q.shape
    Lin = kv.shape[1]

    vt = _value_table(kv.reshape(B * Lin, C), w_norm2, b_norm2, W_val.T, b_val)

    idx256, w256 = _q_projections(q.reshape(B * Lq, C),
                                  reference_points.reshape(B * Lq, 2),
                                  w_norm1, b_norm1,
                                  W_off, b_off, W_attn, b_attn, blk=512)
    nrows = B * Lq * NH
    idx = idx256[:, :192].reshape(nrows, 16)
    wts = w256[:, :192].reshape(nrows, 16)

    attn2d = _sc_gather(vt, idx, wts, nrows).reshape(B * Lq, C)

    out = _out_proj(attn2d, q.reshape(B * Lq, C), W_out, b_out, gamma, blk=512)
    return out.reshape(B, Lq, C)
